# baseline (speedup 1.0000x reference)
import functools

import jax
import jax.numpy as jnp
from jax.experimental import pallas as pl
from jax.experimental.pallas import tpu as pltpu

A = 4
K = 100000
D_OBS = 512
D_HID = 512
D_KEY = 128
P = 50
DELTA = 1e-3
B = 1024

KP = 100352
KB = 2048
NK = KP // KB
BT = 128
NB = B // BT
PAD_KEY = 1e4
MAX_ITERS = 40
BM = 256


def _mlp_body(obs_ref, w1_ref, b1_ref, w2_ref, b2_ref, out_ref):
    h = jnp.dot(obs_ref[...], w1_ref[...], preferred_element_type=jnp.float32)
    h = jnp.maximum(h + b1_ref[...], 0.0)
    out_ref[...] = jnp.dot(h, w2_ref[...], preferred_element_type=jnp.float32) + b2_ref[...]


def _rep(x):
    return jax.lax.broadcast_in_dim(x, (BT, 128), (0, 1)) + jnp.zeros(
        (BT, 128), jnp.float32)


BIG = 3.0e38


def _dnd_body(keys_ref, mk_ref, v_ref, out_ref, d2_ref, mn_ref, mx_ref):
    kb = pl.program_id(2)
    q = keys_ref[...]
    mk = mk_ref[0]
    prod = jax.lax.dot_general(
        q, mk, (((1,), (1,)), ((), ())), preferred_element_type=jnp.float32
    )
    q2 = jnp.sum(q * q, axis=1, keepdims=True)
    m2 = jnp.sum(mk * mk, axis=1)
    d2b = q2 - 2.0 * prod + m2[None, :]
    d2_ref[:, pl.ds(kb * KB, KB)] = d2b

    @pl.when(kb == 0)
    def _init_mm():
        mn_ref[...] = jnp.full((BT, 128), BIG)
        mx_ref[...] = jnp.full((BT, 128), -BIG)

    col = jax.lax.broadcasted_iota(jnp.int32, (BT, KB), 1) + kb * KB
    mn_ref[...] = jnp.minimum(
        mn_ref[...], _rep(jnp.min(d2b, axis=1, keepdims=True)))
    mx_ref[...] = jnp.maximum(
        mx_ref[...], _rep(jnp.max(jnp.where(col < K, d2b, -BIG),
                                  axis=1, keepdims=True)))

    @pl.when(kb == NK - 1)
    def _select_and_reduce():
        rep = _rep

        def chunk(i):
            return d2_ref[:, pl.ds(i * KB, KB)]

        lo0 = mn_ref[...] - 1.0
        hi0 = mx_ref[...]
        clo0 = jnp.zeros((BT, 128))
        chi0 = jnp.full((BT, 128), float(K))
        done0 = jnp.zeros((BT, 128), dtype=jnp.float32)

        def cond(state):
            it, _, _, _, _, done = state
            return jnp.logical_and(it < MAX_ITERS, jnp.min(done) < 0.5)

        def body(state):
            it, lo, hi, clo, chi, done = state
            span = hi - lo
            mid_bi = lo + 0.5 * span
            frac = (float(P) - clo) / jnp.maximum(chi - clo, 1.0)
            mid_in = jnp.clip(lo + frac * span,
                              lo + 0.02 * span, hi - 0.02 * span)
            mid = jnp.where((it % 2) == 0, mid_bi, mid_in)

            def cstep(i, acc):
                blk = chunk(i)
                return acc + rep(jnp.sum(
                    jnp.where(blk <= mid[:, 0:1], 1.0, 0.0),
                    axis=1, keepdims=True))

            cnt = jax.lax.fori_loop(0, NK, cstep, jnp.zeros((BT, 128)))
            ge = cnt >= float(P)
            pend = done < 0.5
            up_hi = jnp.logical_and(ge, pend)
            up_lo = jnp.logical_and(jnp.logical_not(ge), pend)
            new_hi = jnp.where(up_hi, mid, hi)
            new_chi = jnp.where(up_hi, cnt, chi)
            new_lo = jnp.where(up_lo, mid, lo)
            new_clo = jnp.where(up_lo, cnt, clo)
            new_done = jnp.maximum(done, jnp.where(cnt == float(P), 1.0, 0.0))
            return it + 1, new_lo, new_hi, new_clo, new_chi, new_done

        _, _, t, _, _, _ = jax.lax.while_loop(
            cond, body, (0, lo0, hi0, clo0, chi0, done0))

        def fstep(i, c):
            nm, dn = c
            blk = chunk(i)
            w = jnp.where(blk <= t[:, 0:1], 1.0 / (blk + DELTA), 0.0)
            v = v_ref[0, :, pl.ds(i * KB, KB)]
            nm = nm + rep(jnp.sum(w * v, axis=1, keepdims=True))
            dn = dn + rep(jnp.sum(w, axis=1, keepdims=True))
            return nm, dn

        num, den = jax.lax.fori_loop(
            0, NK, fstep, (jnp.zeros((BT, 128)), jnp.zeros((BT, 128))))
        out_ref[...] = (num[:, 0:1] / den[:, 0:1]).reshape(1, 1, BT, 1)


@jax.jit
def kernel(observations, W1, b1, W2, b2, dnd_keys, dnd_values):
    keys = pl.pallas_call(
        _mlp_body,
        grid=(B // BM,),
        in_specs=[
            pl.BlockSpec((BM, D_OBS), lambda i: (i, 0)),
            pl.BlockSpec((D_OBS, D_HID), lambda i: (0, 0)),
            pl.BlockSpec((1, D_HID), lambda i: (0, 0)),
            pl.BlockSpec((D_HID, D_KEY), lambda i: (0, 0)),
            pl.BlockSpec((1, D_KEY), lambda i: (0, 0)),
        ],
        out_specs=pl.BlockSpec((BM, D_KEY), lambda i: (i, 0)),
        out_shape=jax.ShapeDtypeStruct((B, D_KEY), jnp.float32),
    )(observations, W1, b1[None, :], W2, b2[None, :])

    mk_pad = jnp.pad(dnd_keys, ((0, 0), (0, KP - K), (0, 0)),
                     constant_values=PAD_KEY)
    v_pad = jnp.pad(dnd_values, ((0, 0), (0, KP - K)))

    out = pl.pallas_call(
        _dnd_body,
        grid=(A, NB, NK),
        in_specs=[
            pl.BlockSpec((BT, D_KEY), lambda a, bt, kb: (bt, 0)),
            pl.BlockSpec((1, KB, D_KEY), lambda a, bt, kb: (a, kb, 0)),
            pl.BlockSpec((1, 1, KP), lambda a, bt, kb: (a, 0, 0)),
        ],
        out_specs=pl.BlockSpec((1, 1, BT, 1), lambda a, bt, kb: (a, bt, 0, 0)),
        out_shape=jax.ShapeDtypeStruct((A, NB, BT, 1), jnp.float32),
        scratch_shapes=[pltpu.VMEM((BT, KP), jnp.float32),
                        pltpu.VMEM((BT, 128), jnp.float32),
                        pltpu.VMEM((BT, 128), jnp.float32)],
    )(keys, mk_pad, v_pad[:, None, :])

    return out.reshape(A, B).T


# speedup vs baseline: 1.1413x; 1.1413x over previous
import functools

import jax
import jax.numpy as jnp
from jax.experimental import pallas as pl
from jax.experimental.pallas import tpu as pltpu

A = 4
K = 100000
D_OBS = 512
D_HID = 512
D_KEY = 128
P = 50
DELTA = 1e-3
B = 1024

KP = 100352
KB = 2048
NK = KP // KB
BT = 128
NB = B // BT
PAD_KEY = 1e4
MAX_ITERS = 40
BM = 256


def _mlp_body(obs_ref, w1_ref, b1_ref, w2_ref, b2_ref, out_ref):
    h = jnp.dot(obs_ref[...], w1_ref[...], preferred_element_type=jnp.float32)
    h = jnp.maximum(h + b1_ref[...], 0.0)
    out_ref[...] = jnp.dot(h, w2_ref[...], preferred_element_type=jnp.float32) + b2_ref[...]


def _rep(x):
    return jax.lax.broadcast_in_dim(x, (BT, 128), (0, 1)) + jnp.zeros(
        (BT, 128), jnp.float32)


BIG = 3.0e38


def _dnd_body(keys_ref, mk_ref, v_ref, out_ref, d2_ref, mn_ref, mx_ref):
    kb = pl.program_id(2)
    q = keys_ref[...]
    mk = mk_ref[0]
    prod = jax.lax.dot_general(
        q, mk, (((1,), (1,)), ((), ())), preferred_element_type=jnp.float32
    )
    q2 = jnp.sum(q * q, axis=1, keepdims=True)
    m2 = jnp.sum(mk * mk, axis=1)
    d2b = q2 - 2.0 * prod + m2[None, :]
    d2_ref[:, pl.ds(kb * KB, KB)] = d2b

    @pl.when(kb == 0)
    def _init_mm():
        mn_ref[...] = jnp.full((BT, 128), BIG)
        mx_ref[...] = jnp.full((BT, 128), -BIG)

    col = jax.lax.broadcasted_iota(jnp.int32, (BT, KB), 1) + kb * KB
    mn_ref[...] = jnp.minimum(
        mn_ref[...], _rep(jnp.min(d2b, axis=1, keepdims=True)))
    mx_ref[...] = jnp.maximum(
        mx_ref[...], _rep(jnp.max(jnp.where(col < K, d2b, -BIG),
                                  axis=1, keepdims=True)))

    @pl.when(kb == NK - 1)
    def _select_and_reduce():
        rep = _rep

        def chunk(i):
            return d2_ref[:, pl.ds(i * KB, KB)]

        lo0 = mn_ref[...] - 1.0
        hi0 = mx_ref[...]
        clo0 = jnp.zeros((BT, 128))
        chi0 = jnp.full((BT, 128), float(K))
        done0 = jnp.zeros((BT, 128), dtype=jnp.float32)

        def cond(state):
            it, _, _, _, _, done = state
            return jnp.logical_and(it < MAX_ITERS, jnp.min(done) < 0.5)

        def body(state):
            it, lo, hi, clo, chi, done = state
            mid = 0.5 * (lo + hi)

            def cstep(i, acc):
                blk = chunk(i)
                return acc + rep(jnp.sum(
                    jnp.where(blk <= mid[:, 0:1], 1.0, 0.0),
                    axis=1, keepdims=True))

            cnt = jax.lax.fori_loop(0, NK, cstep, jnp.zeros((BT, 128)))
            ge = cnt >= float(P)
            pend = done < 0.5
            up_hi = jnp.logical_and(ge, pend)
            up_lo = jnp.logical_and(jnp.logical_not(ge), pend)
            new_hi = jnp.where(up_hi, mid, hi)
            new_chi = jnp.where(up_hi, cnt, chi)
            new_lo = jnp.where(up_lo, mid, lo)
            new_clo = jnp.where(up_lo, cnt, clo)
            new_done = jnp.maximum(done, jnp.where(cnt == float(P), 1.0, 0.0))
            return it + 1, new_lo, new_hi, new_clo, new_chi, new_done

        _, _, t, _, _, _ = jax.lax.while_loop(
            cond, body, (0, lo0, hi0, clo0, chi0, done0))

        def fstep(i, c):
            nm, dn = c
            blk = chunk(i)
            w = jnp.where(blk <= t[:, 0:1], 1.0 / (blk + DELTA), 0.0)
            v = v_ref[0, :, pl.ds(i * KB, KB)]
            nm = nm + rep(jnp.sum(w * v, axis=1, keepdims=True))
            dn = dn + rep(jnp.sum(w, axis=1, keepdims=True))
            return nm, dn

        num, den = jax.lax.fori_loop(
            0, NK, fstep, (jnp.zeros((BT, 128)), jnp.zeros((BT, 128))))
        out_ref[...] = (num[:, 0:1] / den[:, 0:1]).reshape(1, 1, BT, 1)


@jax.jit
def kernel(observations, W1, b1, W2, b2, dnd_keys, dnd_values):
    keys = pl.pallas_call(
        _mlp_body,
        grid=(B // BM,),
        in_specs=[
            pl.BlockSpec((BM, D_OBS), lambda i: (i, 0)),
            pl.BlockSpec((D_OBS, D_HID), lambda i: (0, 0)),
            pl.BlockSpec((1, D_HID), lambda i: (0, 0)),
            pl.BlockSpec((D_HID, D_KEY), lambda i: (0, 0)),
            pl.BlockSpec((1, D_KEY), lambda i: (0, 0)),
        ],
        out_specs=pl.BlockSpec((BM, D_KEY), lambda i: (i, 0)),
        out_shape=jax.ShapeDtypeStruct((B, D_KEY), jnp.float32),
    )(observations, W1, b1[None, :], W2, b2[None, :])

    mk_pad = jnp.pad(dnd_keys, ((0, 0), (0, KP - K), (0, 0)),
                     constant_values=PAD_KEY)
    v_pad = jnp.pad(dnd_values, ((0, 0), (0, KP - K)))

    out = pl.pallas_call(
        _dnd_body,
        grid=(A, NB, NK),
        in_specs=[
            pl.BlockSpec((BT, D_KEY), lambda a, bt, kb: (bt, 0)),
            pl.BlockSpec((1, KB, D_KEY), lambda a, bt, kb: (a, kb, 0)),
            pl.BlockSpec((1, 1, KP), lambda a, bt, kb: (a, 0, 0)),
        ],
        out_specs=pl.BlockSpec((1, 1, BT, 1), lambda a, bt, kb: (a, bt, 0, 0)),
        out_shape=jax.ShapeDtypeStruct((A, NB, BT, 1), jnp.float32),
        scratch_shapes=[pltpu.VMEM((BT, KP), jnp.float32),
                        pltpu.VMEM((BT, 128), jnp.float32),
                        pltpu.VMEM((BT, 128), jnp.float32)],
    )(keys, mk_pad, v_pad[:, None, :])

    return out.reshape(A, B).T


# speedup vs baseline: 1.4051x; 1.2311x over previous
import functools

import jax
import jax.numpy as jnp
from jax.experimental import pallas as pl
from jax.experimental.pallas import tpu as pltpu

A = 4
K = 100000
D_OBS = 512
D_HID = 512
D_KEY = 128
P = 50
DELTA = 1e-3
B = 1024

KP = 100352
KB = 2048
NK = KP // KB
BT = 128
NB = B // BT
PAD_KEY = 1e4
MAX_ITERS = 40
BM = 256


def _mlp_body(obs_ref, w1_ref, b1_ref, w2_ref, b2_ref, out_ref):
    h = jnp.dot(obs_ref[...], w1_ref[...], preferred_element_type=jnp.float32)
    h = jnp.maximum(h + b1_ref[...], 0.0)
    out_ref[...] = jnp.dot(h, w2_ref[...], preferred_element_type=jnp.float32) + b2_ref[...]


def _rep(x):
    return jax.lax.broadcast_in_dim(x, (BT, 128), (0, 1)) + jnp.zeros(
        (BT, 128), jnp.float32)


BIG = 3.0e38


def _dnd_body(keys_ref, mk_ref, v_ref, out_ref, d2_ref):
    kb = pl.program_id(2)
    q = keys_ref[...]
    mk = mk_ref[0]
    prod = jax.lax.dot_general(
        q, mk, (((1,), (1,)), ((), ())), preferred_element_type=jnp.float32
    )
    q2 = jnp.sum(q * q, axis=1, keepdims=True)
    m2 = jnp.sum(mk * mk, axis=1)
    d2_ref[:, pl.ds(kb * KB, KB)] = q2 - 2.0 * prod + m2[None, :]

    @pl.when(kb == NK - 1)
    def _select_and_reduce():
        rep = _rep

        def chunk(i):
            return d2_ref[:, pl.ds(i * KB, KB)]

        def mm_step(i, c):
            mn, mx = c
            blk = chunk(i)
            col = jax.lax.broadcasted_iota(jnp.int32, (BT, KB), 1) + i * KB
            mn = jnp.minimum(mn, rep(jnp.min(blk, axis=1, keepdims=True)))
            mx = jnp.maximum(mx, rep(jnp.max(
                jnp.where(col < K, blk, -BIG), axis=1, keepdims=True)))
            return mn, mx

        lo0, hi0 = jax.lax.fori_loop(
            0, NK, mm_step,
            (jnp.full((BT, 128), BIG), jnp.full((BT, 128), -BIG)))
        lo0 = lo0 - 1.0
        done0 = jnp.zeros((BT, 128), dtype=jnp.float32)

        def cond(state):
            it, _, _, done = state
            return jnp.logical_and(it < MAX_ITERS, jnp.min(done) < 0.5)

        def body(state):
            it, lo, hi, done = state
            mid = 0.5 * (lo + hi)

            def cstep(i, acc):
                blk = chunk(i)
                return acc + rep(jnp.sum(
                    jnp.where(blk <= mid[:, 0:1], 1.0, 0.0),
                    axis=1, keepdims=True))

            cnt = jax.lax.fori_loop(0, NK, cstep, jnp.zeros((BT, 128)))
            ge = cnt >= float(P)
            pend = done < 0.5
            new_hi = jnp.where(jnp.logical_and(ge, pend), mid, hi)
            new_lo = jnp.where(jnp.logical_and(jnp.logical_not(ge), pend), mid, lo)
            tied = (new_hi - new_lo) < 6e-8 * jnp.abs(new_hi)
            new_done = jnp.maximum(
                done, jnp.where(jnp.logical_or(cnt == float(P), tied), 1.0, 0.0))
            return it + 1, new_lo, new_hi, new_done

        _, _, t, _ = jax.lax.while_loop(cond, body, (0, lo0, hi0, done0))

        def fstep(i, c):
            nm, dn = c
            blk = chunk(i)
            w = jnp.where(blk <= t[:, 0:1], 1.0 / (blk + DELTA), 0.0)
            v = v_ref[0, :, pl.ds(i * KB, KB)]
            nm = nm + rep(jnp.sum(w * v, axis=1, keepdims=True))
            dn = dn + rep(jnp.sum(w, axis=1, keepdims=True))
            return nm, dn

        num, den = jax.lax.fori_loop(
            0, NK, fstep, (jnp.zeros((BT, 128)), jnp.zeros((BT, 128))))
        out_ref[...] = (num[:, 0:1] / den[:, 0:1]).reshape(1, 1, BT, 1)


@jax.jit
def kernel(observations, W1, b1, W2, b2, dnd_keys, dnd_values):
    keys = pl.pallas_call(
        _mlp_body,
        grid=(B // BM,),
        in_specs=[
            pl.BlockSpec((BM, D_OBS), lambda i: (i, 0)),
            pl.BlockSpec((D_OBS, D_HID), lambda i: (0, 0)),
            pl.BlockSpec((1, D_HID), lambda i: (0, 0)),
            pl.BlockSpec((D_HID, D_KEY), lambda i: (0, 0)),
            pl.BlockSpec((1, D_KEY), lambda i: (0, 0)),
        ],
        out_specs=pl.BlockSpec((BM, D_KEY), lambda i: (i, 0)),
        out_shape=jax.ShapeDtypeStruct((B, D_KEY), jnp.float32),
    )(observations, W1, b1[None, :], W2, b2[None, :])

    mk_pad = jnp.pad(dnd_keys, ((0, 0), (0, KP - K), (0, 0)),
                     constant_values=PAD_KEY)
    v_pad = jnp.pad(dnd_values, ((0, 0), (0, KP - K)))

    out = pl.pallas_call(
        _dnd_body,
        grid=(A, NB, NK),
        in_specs=[
            pl.BlockSpec((BT, D_KEY), lambda a, bt, kb: (bt, 0)),
            pl.BlockSpec((1, KB, D_KEY), lambda a, bt, kb: (a, kb, 0)),
            pl.BlockSpec((1, 1, KP), lambda a, bt, kb: (a, 0, 0)),
        ],
        out_specs=pl.BlockSpec((1, 1, BT, 1), lambda a, bt, kb: (a, bt, 0, 0)),
        out_shape=jax.ShapeDtypeStruct((A, NB, BT, 1), jnp.float32),
        scratch_shapes=[pltpu.VMEM((BT, KP), jnp.float32)],
    )(keys, mk_pad, v_pad[:, None, :])

    return out.reshape(A, B).T


# speedup vs baseline: 1.7397x; 1.2381x over previous
import functools

import jax
import jax.numpy as jnp
from jax.experimental import pallas as pl
from jax.experimental.pallas import tpu as pltpu

A = 4
K = 100000
D_OBS = 512
D_HID = 512
D_KEY = 128
P = 50
DELTA = 1e-3
B = 1024

KP = 102400
KB = 4096
NK = KP // KB
BT = 128
NB = B // BT
PAD_KEY = 1e4
MAX_ITERS = 40
BM = 256


def _mlp_body(obs_ref, w1_ref, b1_ref, w2_ref, b2_ref, out_ref):
    h = jnp.dot(obs_ref[...], w1_ref[...], preferred_element_type=jnp.float32)
    h = jnp.maximum(h + b1_ref[...], 0.0)
    out_ref[...] = jnp.dot(h, w2_ref[...], preferred_element_type=jnp.float32) + b2_ref[...]


def _rep(x):
    return jax.lax.broadcast_in_dim(x, (BT, 128), (0, 1)) + jnp.zeros(
        (BT, 128), jnp.float32)


BIG = 3.0e38


def _dnd_body(keys_ref, mk_ref, v_ref, out_ref, d2_ref):
    kb = pl.program_id(2)
    q = keys_ref[...]
    mk = mk_ref[0]
    prod = jax.lax.dot_general(
        q, mk, (((1,), (1,)), ((), ())), preferred_element_type=jnp.float32
    )
    q2 = jnp.sum(q * q, axis=1, keepdims=True)
    m2 = jnp.sum(mk * mk, axis=1)
    d2_ref[:, pl.ds(kb * KB, KB)] = q2 - 2.0 * prod + m2[None, :]

    @pl.when(kb == NK - 1)
    def _select_and_reduce():
        rep = _rep

        def chunk(i):
            return d2_ref[:, pl.ds(i * KB, KB)]

        def mm_step(i, c):
            mn, mx = c
            blk = chunk(i)
            col = jax.lax.broadcasted_iota(jnp.int32, (BT, KB), 1) + i * KB
            mn = jnp.minimum(mn, rep(jnp.min(blk, axis=1, keepdims=True)))
            mx = jnp.maximum(mx, rep(jnp.max(
                jnp.where(col < K, blk, -BIG), axis=1, keepdims=True)))
            return mn, mx

        lo0, hi0 = jax.lax.fori_loop(
            0, NK, mm_step,
            (jnp.full((BT, 128), BIG), jnp.full((BT, 128), -BIG)))
        lo0 = lo0 - 1.0
        done0 = jnp.zeros((BT, 128), dtype=jnp.float32)

        def cond(state):
            it, _, _, done = state
            return jnp.logical_and(it < MAX_ITERS, jnp.min(done) < 0.5)

        def body(state):
            it, lo, hi, done = state
            mid = 0.5 * (lo + hi)

            def cstep(i, acc):
                blk = chunk(i)
                return acc + rep(jnp.sum(
                    jnp.where(blk <= mid[:, 0:1], 1.0, 0.0),
                    axis=1, keepdims=True))

            cnt = jax.lax.fori_loop(0, NK, cstep, jnp.zeros((BT, 128)))
            ge = cnt >= float(P)
            pend = done < 0.5
            new_hi = jnp.where(jnp.logical_and(ge, pend), mid, hi)
            new_lo = jnp.where(jnp.logical_and(jnp.logical_not(ge), pend), mid, lo)
            tied = (new_hi - new_lo) < 6e-8 * jnp.abs(new_hi)
            new_done = jnp.maximum(
                done, jnp.where(jnp.logical_or(cnt == float(P), tied), 1.0, 0.0))
            return it + 1, new_lo, new_hi, new_done

        _, _, t, _ = jax.lax.while_loop(cond, body, (0, lo0, hi0, done0))

        def fstep(i, c):
            nm, dn = c
            blk = chunk(i)
            w = jnp.where(blk <= t[:, 0:1], 1.0 / (blk + DELTA), 0.0)
            v = v_ref[0, :, pl.ds(i * KB, KB)]
            nm = nm + rep(jnp.sum(w * v, axis=1, keepdims=True))
            dn = dn + rep(jnp.sum(w, axis=1, keepdims=True))
            return nm, dn

        num, den = jax.lax.fori_loop(
            0, NK, fstep, (jnp.zeros((BT, 128)), jnp.zeros((BT, 128))))
        out_ref[...] = (num[:, 0:1] / den[:, 0:1]).reshape(1, 1, BT, 1)


@jax.jit
def kernel(observations, W1, b1, W2, b2, dnd_keys, dnd_values):
    keys = pl.pallas_call(
        _mlp_body,
        grid=(B // BM,),
        in_specs=[
            pl.BlockSpec((BM, D_OBS), lambda i: (i, 0)),
            pl.BlockSpec((D_OBS, D_HID), lambda i: (0, 0)),
            pl.BlockSpec((1, D_HID), lambda i: (0, 0)),
            pl.BlockSpec((D_HID, D_KEY), lambda i: (0, 0)),
            pl.BlockSpec((1, D_KEY), lambda i: (0, 0)),
        ],
        out_specs=pl.BlockSpec((BM, D_KEY), lambda i: (i, 0)),
        out_shape=jax.ShapeDtypeStruct((B, D_KEY), jnp.float32),
    )(observations, W1, b1[None, :], W2, b2[None, :])

    mk_pad = jnp.pad(dnd_keys, ((0, 0), (0, KP - K), (0, 0)),
                     constant_values=PAD_KEY)
    v_pad = jnp.pad(dnd_values, ((0, 0), (0, KP - K)))

    out = pl.pallas_call(
        _dnd_body,
        grid=(A, NB, NK),
        in_specs=[
            pl.BlockSpec((BT, D_KEY), lambda a, bt, kb: (bt, 0)),
            pl.BlockSpec((1, KB, D_KEY), lambda a, bt, kb: (a, kb, 0)),
            pl.BlockSpec((1, 1, KP), lambda a, bt, kb: (a, 0, 0)),
        ],
        out_specs=pl.BlockSpec((1, 1, BT, 1), lambda a, bt, kb: (a, bt, 0, 0)),
        out_shape=jax.ShapeDtypeStruct((A, NB, BT, 1), jnp.float32),
        scratch_shapes=[pltpu.VMEM((BT, KP), jnp.float32)],
    )(keys, mk_pad, v_pad[:, None, :])

    return out.reshape(A, B).T
